# Initial kernel scaffold; baseline (speedup 1.0000x reference)
import jax
import jax.numpy as jnp
from jax.experimental import pallas as pl


def kernel(x, pos_emb):
    raise NotImplementedError("write your pallas kernel here")


# speedup vs baseline: 2.4407x; 2.4407x over previous
import jax
import jax.numpy as jnp
from jax.experimental import pallas as pl

_BLOCK = 512


def _rev_block(in_ref, out_ref):
    b = in_ref.shape[0]
    rows = jax.lax.broadcasted_iota(jnp.int32, (b, b), 0)
    cols = jax.lax.broadcasted_iota(jnp.int32, (b, b), 1)
    perm = (rows + cols == b - 1).astype(jnp.float32)
    out_ref[:] = jnp.dot(perm, in_ref[:], preferred_element_type=jnp.float32)


def kernel(x, pos_emb):
    maxlen = x.shape[-1]
    dim = pos_emb.shape[1]
    num_blocks = maxlen // _BLOCK
    return pl.pallas_call(
        _rev_block,
        grid=(num_blocks,),
        in_specs=[
            pl.BlockSpec((_BLOCK, dim), lambda i: (num_blocks - 1 - i, 0)),
        ],
        out_specs=pl.BlockSpec((_BLOCK, dim), lambda i: (i, 0)),
        out_shape=jax.ShapeDtypeStruct((maxlen, dim), pos_emb.dtype),
    )(pos_emb[:maxlen])


# speedup vs baseline: 3.2034x; 1.3125x over previous
import jax
import jax.numpy as jnp
from jax.experimental import pallas as pl

_BLOCK = 1024


def _rev_block(in_ref, out_ref):
    b = in_ref.shape[0]
    rows = jax.lax.broadcasted_iota(jnp.int32, (b, b), 0)
    cols = jax.lax.broadcasted_iota(jnp.int32, (b, b), 1)
    perm = (rows + cols == b - 1).astype(jnp.float32)
    out_ref[:] = jnp.dot(perm, in_ref[:], preferred_element_type=jnp.float32)


def kernel(x, pos_emb):
    maxlen = x.shape[-1]
    dim = pos_emb.shape[1]
    num_blocks = maxlen // _BLOCK
    return pl.pallas_call(
        _rev_block,
        grid=(num_blocks,),
        in_specs=[
            pl.BlockSpec((_BLOCK, dim), lambda i: (num_blocks - 1 - i, 0)),
        ],
        out_specs=pl.BlockSpec((_BLOCK, dim), lambda i: (i, 0)),
        out_shape=jax.ShapeDtypeStruct((maxlen, dim), pos_emb.dtype),
    )(pos_emb[:maxlen])
